# Initial kernel scaffold; baseline (speedup 1.0000x reference)
import jax
import jax.numpy as jnp
from jax.experimental import pallas as pl


def kernel(x, edge_index, W1, b1, W2, b2):
    raise NotImplementedError("write your pallas kernel here")


# speedup vs baseline: 20.3116x; 20.3116x over previous
import functools

import jax
import jax.numpy as jnp
from jax import lax
from jax.experimental import pallas as pl
from jax.experimental.pallas import tpu as pltpu
from jax.experimental.pallas import tpu_sc as plsc

N = 10000
D = 128
E = 320000
NC = 2
NS = 16
NW = NC * NS
EPW = E // NW
C = 80
CH = EPW // C
NP = 10240
RPS = NP // NS

_MESH = plsc.VectorSubcoreMesh(core_axis_name="c", subcore_axis_name="s")


@functools.partial(
    pl.kernel,
    out_type=jax.ShapeDtypeStruct((NC, NP), jnp.float32),
    mesh=_MESH,
    scratch_types=[
        pltpu.VMEM((CH, C), jnp.int32),
        pltpu.VMEM((C,), jnp.float32),
        pltpu.VMEM((RPS,), jnp.float32),
        pltpu.VMEM_SHARED((NP,), jnp.float32),
    ],
)
def _deg_sc(dst_hbm, out_hbm, dst_v, ones_v, z_v, acc_sh):
    cid = lax.axis_index("c")
    sid = lax.axis_index("s")
    wid = sid * NC + cid
    one16 = jnp.ones((16,), jnp.float32)
    zero16 = jnp.zeros((16,), jnp.float32)
    for k in range(C // 16):
        ones_v[pl.ds(k * 16, 16)] = one16

    def _zb(i, carry):
        z_v[pl.ds(i * 16, 16)] = zero16
        return carry

    lax.fori_loop(0, RPS // 16, _zb, 0)
    pltpu.sync_copy(z_v, acc_sh.at[pl.ds(sid * RPS, RPS)])
    pltpu.sync_copy(dst_hbm.at[wid], dst_v)
    plsc.subcore_barrier()

    def _body(j, carry):
        pltpu.sync_copy(ones_v, acc_sh.at[dst_v.at[j]], add=True)
        return carry

    lax.fori_loop(0, CH, _body, 0)
    plsc.subcore_barrier()
    pltpu.sync_copy(acc_sh.at[pl.ds(sid * RPS, RPS)],
                    out_hbm.at[cid, pl.ds(sid * RPS, RPS)])


@functools.partial(
    pl.kernel,
    out_type=jax.ShapeDtypeStruct((NC, NP, D), jnp.float32),
    mesh=_MESH,
    scratch_types=[
        pltpu.VMEM((CH, C), jnp.int32),
        pltpu.VMEM((CH, C), jnp.int32),
        pltpu.VMEM((C, D), jnp.float32),
        pltpu.VMEM_SHARED((NP, D), jnp.float32),
        pltpu.SemaphoreType.DMA,
    ],
)
def _scat_sc(g_hbm, src_hbm, dst_hbm, out_hbm, src_v, dst_v, rows_v,
             acc_sh, sem):
    cid = lax.axis_index("c")
    sid = lax.axis_index("s")
    wid = sid * NC + cid
    zero16 = jnp.zeros((16,), jnp.float32)

    def _zrow(i, carry):
        for k in range(D // 16):
            rows_v[i, pl.ds(k * 16, 16)] = zero16
        return carry

    lax.fori_loop(0, C, _zrow, 0)
    for t in range(RPS // C):
        pltpu.sync_copy(rows_v, acc_sh.at[pl.ds(sid * RPS + t * C, C)])
    pltpu.sync_copy(src_hbm.at[wid], src_v)
    pltpu.sync_copy(dst_hbm.at[wid], dst_v)
    plsc.subcore_barrier()

    def _body(j, carry):
        pltpu.async_copy(g_hbm.at[src_v.at[j]], rows_v, sem).wait()
        pltpu.sync_copy(rows_v, acc_sh.at[dst_v.at[j]], add=True)
        return carry

    lax.fori_loop(0, CH, _body, 0)
    plsc.subcore_barrier()
    pltpu.sync_copy(acc_sh.at[pl.ds(sid * RPS, RPS)],
                    out_hbm.at[cid, pl.ds(sid * RPS, RPS)])


R = 2000
GRID = N // R


def _dis_from(degT_ref):
    d = degT_ref[:, 0:1] + degT_ref[:, 1:2] + 1.0
    return lax.rsqrt(d)


def _lin_in_body(x_ref, degT_ref, w_ref, g_ref):
    dis = _dis_from(degT_ref)
    h = jnp.dot(x_ref[...], w_ref[...], preferred_element_type=jnp.float32)
    g_ref[...] = dis * h


_lin_in = pl.pallas_call(
    _lin_in_body,
    grid=(GRID,),
    in_specs=[
        pl.BlockSpec((R, D), lambda i: (i, 0)),
        pl.BlockSpec((R, 2), lambda i: (i, 0)),
        pl.BlockSpec((D, D), lambda i: (0, 0)),
    ],
    out_specs=pl.BlockSpec((R, D), lambda i: (i, 0)),
    out_shape=jax.ShapeDtypeStruct((N, D), jnp.float32),
)


def _mid_body(acc_ref, g1_ref, degT_ref, b1_ref, w_ref, g2_ref):
    dis = _dis_from(degT_ref)
    agg = acc_ref[0] + acc_ref[1] + g1_ref[...]
    h1 = jnp.maximum(dis * agg + b1_ref[...], 0.0)
    h2 = jnp.dot(h1, w_ref[...], preferred_element_type=jnp.float32)
    g2_ref[...] = dis * h2


_mid = pl.pallas_call(
    _mid_body,
    grid=(GRID,),
    in_specs=[
        pl.BlockSpec((NC, R, D), lambda i: (0, i, 0)),
        pl.BlockSpec((R, D), lambda i: (i, 0)),
        pl.BlockSpec((R, 2), lambda i: (i, 0)),
        pl.BlockSpec((1, D), lambda i: (0, 0)),
        pl.BlockSpec((D, D), lambda i: (0, 0)),
    ],
    out_specs=pl.BlockSpec((R, D), lambda i: (i, 0)),
    out_shape=jax.ShapeDtypeStruct((N, D), jnp.float32),
)


def _fin_body(acc_ref, g2_ref, degT_ref, b2_ref, out_ref):
    dis = _dis_from(degT_ref)
    agg = acc_ref[0] + acc_ref[1] + g2_ref[...]
    out_ref[...] = dis * agg + b2_ref[...]


_fin = pl.pallas_call(
    _fin_body,
    grid=(GRID,),
    in_specs=[
        pl.BlockSpec((NC, R, D), lambda i: (0, i, 0)),
        pl.BlockSpec((R, D), lambda i: (i, 0)),
        pl.BlockSpec((R, 2), lambda i: (i, 0)),
        pl.BlockSpec((1, D), lambda i: (0, 0)),
    ],
    out_specs=pl.BlockSpec((R, D), lambda i: (i, 0)),
    out_shape=jax.ShapeDtypeStruct((N, D), jnp.float32),
)


def kernel(x, edge_index, W1, b1, W2, b2):
    src3 = edge_index[0].reshape(NW, CH, C)
    dst3 = edge_index[1].reshape(NW, CH, C)
    degp = _deg_sc(dst3)
    degT = degp.T
    g1 = _lin_in(x, degT, W1)
    acc1 = _scat_sc(g1, src3, dst3)
    g2 = _mid(acc1, g1, degT, b1.reshape(1, D), W2)
    acc2 = _scat_sc(g2, src3, dst3)
    out = _fin(acc2, g2, degT, b2.reshape(1, D))
    return out


# speedup vs baseline: 25.8454x; 1.2724x over previous
import functools

import jax
import jax.numpy as jnp
from jax import lax
from jax.experimental import pallas as pl
from jax.experimental.pallas import tpu as pltpu
from jax.experimental.pallas import tpu_sc as plsc

N = 10000
D = 128
E = 320000
NC = 2
NS = 16
NW = NC * NS
EPW = E // NW
C = 80
CH = EPW // C
NP = 10240
RPS = NP // NS

_MESH = plsc.VectorSubcoreMesh(core_axis_name="c", subcore_axis_name="s")


@functools.partial(
    pl.kernel,
    out_type=jax.ShapeDtypeStruct((NC, NP), jnp.float32),
    mesh=_MESH,
    scratch_types=[
        pltpu.VMEM((CH, C), jnp.int32),
        pltpu.VMEM((C,), jnp.float32),
        pltpu.VMEM((RPS,), jnp.float32),
        pltpu.VMEM_SHARED((NP,), jnp.float32),
    ],
)
def _deg_sc(dst_hbm, out_hbm, dst_v, ones_v, z_v, acc_sh):
    cid = lax.axis_index("c")
    sid = lax.axis_index("s")
    wid = sid * NC + cid
    one16 = jnp.ones((16,), jnp.float32)
    zero16 = jnp.zeros((16,), jnp.float32)
    for k in range(C // 16):
        ones_v[pl.ds(k * 16, 16)] = one16

    def _zb(i, carry):
        z_v[pl.ds(i * 16, 16)] = zero16
        return carry

    lax.fori_loop(0, RPS // 16, _zb, 0)
    pltpu.sync_copy(z_v, acc_sh.at[pl.ds(sid * RPS, RPS)])
    pltpu.sync_copy(dst_hbm.at[wid], dst_v)
    plsc.subcore_barrier()

    def _body(j, carry):
        pltpu.sync_copy(ones_v, acc_sh.at[dst_v.at[j]], add=True)
        return carry

    lax.fori_loop(0, CH, _body, 0)
    plsc.subcore_barrier()
    pltpu.sync_copy(acc_sh.at[pl.ds(sid * RPS, RPS)],
                    out_hbm.at[cid, pl.ds(sid * RPS, RPS)])


@functools.partial(
    pl.kernel,
    out_type=jax.ShapeDtypeStruct((NC, NP, D), jnp.float32),
    mesh=_MESH,
    scratch_types=[
        pltpu.VMEM((EPW,), jnp.int32),
        pltpu.VMEM((CH, C), jnp.int32),
        pltpu.VMEM((C, D), jnp.float32),
        pltpu.VMEM((C, D), jnp.float32),
        pltpu.VMEM_SHARED((NP, D), jnp.float32),
        pltpu.SemaphoreType.DMA,
        pltpu.SemaphoreType.DMA,
        pltpu.SemaphoreType.DMA,
        pltpu.SemaphoreType.DMA,
    ],
)
def _scat_sc(g_hbm, src_hbm, dst_hbm, out_hbm, src_v, dst_v, rows0, rows1,
             acc_sh, gs0, gs1, ss0, ss1):
    cid = lax.axis_index("c")
    sid = lax.axis_index("s")
    wid = sid * NC + cid
    zero16 = jnp.zeros((16,), jnp.float32)

    def _zrow(i, carry):
        for k in range(D // 16):
            rows0[i, pl.ds(k * 16, 16)] = zero16
        return carry

    lax.fori_loop(0, C, _zrow, 0)
    for t in range(RPS // C):
        pltpu.sync_copy(rows0, acc_sh.at[pl.ds(sid * RPS + t * C, C)])
    pltpu.sync_copy(src_hbm.at[wid], src_v)
    pltpu.sync_copy(dst_hbm.at[wid], dst_v)
    plsc.subcore_barrier()

    def _gissue(j, buf, sem):
        pltpu.async_copy(g_hbm.at[src_v.at[pl.ds(j * C, C)]], buf, sem)

    def _gwait(buf, sem):
        pltpu.make_async_copy(g_hbm.at[pl.ds(0, C)], buf, sem).wait()

    def _sissue(j, buf, sem):
        pltpu.async_copy(buf, acc_sh.at[dst_v.at[j]], sem, add=True)

    def _swait(buf, sem):
        pltpu.make_async_copy(buf, acc_sh.at[pl.ds(0, C)], sem).wait()

    _gissue(0, rows0, gs0)
    _gissue(1, rows1, gs1)

    def _body(t, carry):
        j0 = 2 * t
        j1 = j0 + 1
        j2 = lax.rem(j0 + 2, CH)
        j3 = lax.rem(j0 + 3, CH)
        _gwait(rows0, gs0)
        _sissue(j0, rows0, ss0)
        _gwait(rows1, gs1)
        _sissue(j1, rows1, ss1)
        _swait(rows0, ss0)
        _gissue(j2, rows0, gs0)
        _swait(rows1, ss1)
        _gissue(j3, rows1, gs1)
        return carry

    lax.fori_loop(0, CH // 2, _body, 0)
    _gwait(rows0, gs0)
    _sissue(CH - 1, rows0, ss0)
    _swait(rows0, ss0)
    _gwait(rows1, gs1)
    plsc.subcore_barrier()
    pltpu.sync_copy(acc_sh.at[pl.ds(sid * RPS, RPS)],
                    out_hbm.at[cid, pl.ds(sid * RPS, RPS)])


R = 2000
GRID = N // R


def _dis_from(degT_ref):
    d = degT_ref[:, 0:1] + degT_ref[:, 1:2] + 1.0
    return lax.rsqrt(d)


def _lin_in_body(x_ref, degT_ref, w_ref, g_ref):
    dis = _dis_from(degT_ref)
    h = jnp.dot(x_ref[...], w_ref[...], preferred_element_type=jnp.float32)
    g_ref[...] = dis * h


_lin_in = pl.pallas_call(
    _lin_in_body,
    grid=(GRID,),
    in_specs=[
        pl.BlockSpec((R, D), lambda i: (i, 0)),
        pl.BlockSpec((R, 2), lambda i: (i, 0)),
        pl.BlockSpec((D, D), lambda i: (0, 0)),
    ],
    out_specs=pl.BlockSpec((R, D), lambda i: (i, 0)),
    out_shape=jax.ShapeDtypeStruct((N, D), jnp.float32),
)


def _mid_body(acc_ref, g1_ref, degT_ref, b1_ref, w_ref, g2_ref):
    dis = _dis_from(degT_ref)
    agg = acc_ref[0] + acc_ref[1] + g1_ref[...]
    h1 = jnp.maximum(dis * agg + b1_ref[...], 0.0)
    h2 = jnp.dot(h1, w_ref[...], preferred_element_type=jnp.float32)
    g2_ref[...] = dis * h2


_mid = pl.pallas_call(
    _mid_body,
    grid=(GRID,),
    in_specs=[
        pl.BlockSpec((NC, R, D), lambda i: (0, i, 0)),
        pl.BlockSpec((R, D), lambda i: (i, 0)),
        pl.BlockSpec((R, 2), lambda i: (i, 0)),
        pl.BlockSpec((1, D), lambda i: (0, 0)),
        pl.BlockSpec((D, D), lambda i: (0, 0)),
    ],
    out_specs=pl.BlockSpec((R, D), lambda i: (i, 0)),
    out_shape=jax.ShapeDtypeStruct((N, D), jnp.float32),
)


def _fin_body(acc_ref, g2_ref, degT_ref, b2_ref, out_ref):
    dis = _dis_from(degT_ref)
    agg = acc_ref[0] + acc_ref[1] + g2_ref[...]
    out_ref[...] = dis * agg + b2_ref[...]


_fin = pl.pallas_call(
    _fin_body,
    grid=(GRID,),
    in_specs=[
        pl.BlockSpec((NC, R, D), lambda i: (0, i, 0)),
        pl.BlockSpec((R, D), lambda i: (i, 0)),
        pl.BlockSpec((R, 2), lambda i: (i, 0)),
        pl.BlockSpec((1, D), lambda i: (0, 0)),
    ],
    out_specs=pl.BlockSpec((R, D), lambda i: (i, 0)),
    out_shape=jax.ShapeDtypeStruct((N, D), jnp.float32),
)


def kernel(x, edge_index, W1, b1, W2, b2):
    src3 = edge_index[0].reshape(NW, EPW)
    dst3 = edge_index[1].reshape(NW, CH, C)
    degp = _deg_sc(dst3)
    degT = degp.T
    g1 = _lin_in(x, degT, W1)
    acc1 = _scat_sc(g1, src3, dst3)
    g2 = _mid(acc1, g1, degT, b1.reshape(1, D), W2)
    acc2 = _scat_sc(g2, src3, dst3)
    out = _fin(acc2, g2, degT, b2.reshape(1, D))
    return out


# speedup vs baseline: 31.2038x; 1.2073x over previous
import functools

import jax
import jax.numpy as jnp
from jax import lax
from jax.experimental import pallas as pl
from jax.experimental.pallas import tpu as pltpu
from jax.experimental.pallas import tpu_sc as plsc

N = 10000
D = 128
E = 320000
NC = 2
NS = 16
NW = NC * NS
EPW = E // NW
C = 80
CH = EPW // C
NB = 3
NP = 10240
RPS = NP // NS

_MESH = plsc.VectorSubcoreMesh(core_axis_name="c", subcore_axis_name="s")


@functools.partial(
    pl.kernel,
    out_type=jax.ShapeDtypeStruct((NC, NP), jnp.float32),
    mesh=_MESH,
    scratch_types=[
        pltpu.VMEM((CH, C), jnp.int32),
        pltpu.VMEM((C,), jnp.float32),
        pltpu.VMEM((RPS,), jnp.float32),
        pltpu.VMEM_SHARED((NP,), jnp.float32),
    ],
)
def _deg_sc(dst_hbm, out_hbm, dst_v, ones_v, z_v, acc_sh):
    cid = lax.axis_index("c")
    sid = lax.axis_index("s")
    wid = sid * NC + cid
    one16 = jnp.ones((16,), jnp.float32)
    zero16 = jnp.zeros((16,), jnp.float32)
    for k in range(C // 16):
        ones_v[pl.ds(k * 16, 16)] = one16

    def _zb(i, carry):
        z_v[pl.ds(i * 16, 16)] = zero16
        return carry

    lax.fori_loop(0, RPS // 16, _zb, 0)
    pltpu.sync_copy(z_v, acc_sh.at[pl.ds(sid * RPS, RPS)])
    pltpu.sync_copy(dst_hbm.at[wid], dst_v)
    plsc.subcore_barrier()

    def _body(j, carry):
        pltpu.sync_copy(ones_v, acc_sh.at[dst_v.at[j]], add=True)
        return carry

    lax.fori_loop(0, CH, _body, 0)
    plsc.subcore_barrier()
    pltpu.sync_copy(acc_sh.at[pl.ds(sid * RPS, RPS)],
                    out_hbm.at[cid, pl.ds(sid * RPS, RPS)])


@functools.partial(
    pl.kernel,
    out_type=jax.ShapeDtypeStruct((NC, NP, D), jnp.float32),
    mesh=_MESH,
    scratch_types=[
        pltpu.VMEM((EPW,), jnp.int32),
        pltpu.VMEM((NB, C), jnp.int32),
        pltpu.VMEM((NB, C, D), jnp.float32),
        pltpu.VMEM_SHARED((NP, D), jnp.float32),
        [pltpu.SemaphoreType.DMA] * NB,
        [pltpu.SemaphoreType.DMA] * NB,
        [pltpu.SemaphoreType.DMA] * NB,
    ],
)
def _scat_sc(g_hbm, src_hbm, dst_hbm, out_hbm, src_v, didx, rows,
             acc_sh, gsem, ssem, dsem):
    cid = lax.axis_index("c")
    sid = lax.axis_index("s")
    wid = sid * NC + cid
    zero16 = jnp.zeros((16,), jnp.float32)

    def _zrow(i, carry):
        for k in range(D // 16):
            rows[0, i, pl.ds(k * 16, 16)] = zero16
        return carry

    lax.fori_loop(0, C, _zrow, 0)
    for t in range(RPS // C):
        pltpu.sync_copy(rows.at[0], acc_sh.at[pl.ds(sid * RPS + t * C, C)])
    pltpu.sync_copy(src_hbm.at[wid], src_v)
    plsc.subcore_barrier()

    def _gissue(j, b):
        pltpu.async_copy(g_hbm.at[src_v.at[pl.ds(j * C, C)]], rows.at[b],
                         gsem[b])

    def _gwait(b):
        pltpu.make_async_copy(g_hbm.at[pl.ds(0, C)], rows.at[b],
                              gsem[b]).wait()

    def _dissue(j, b):
        pltpu.async_copy(dst_hbm.at[wid, j], didx.at[b], dsem[b])

    def _dwait(b):
        pltpu.make_async_copy(dst_hbm.at[0, 0], didx.at[b], dsem[b]).wait()

    def _sissue(b):
        pltpu.async_copy(rows.at[b], acc_sh.at[didx.at[b]], ssem[b],
                         add=True)

    def _swait(b):
        pltpu.make_async_copy(rows.at[b], acc_sh.at[pl.ds(0, C)],
                              ssem[b]).wait()

    for b in range(NB):
        _dissue(b, b)
        _gissue(b, b)

    def _body(t, carry):
        j = NB * t
        for b in range(NB):
            _gwait(b)
            _dwait(b)
            _sissue(b)
        for b in range(NB):
            jn = lax.rem(j + b + NB, CH)
            _swait(b)
            _dissue(jn, b)
            _gissue(jn, b)
        return carry

    lax.fori_loop(0, CH // NB, _body, 0)
    for b in range(CH % NB):
        _gwait(b)
        _dwait(b)
        _sissue(b)
    for b in range(CH % NB):
        _swait(b)
    for b in range(CH % NB, NB):
        _gwait(b)
        _dwait(b)
    plsc.subcore_barrier()
    pltpu.sync_copy(acc_sh.at[pl.ds(sid * RPS, RPS)],
                    out_hbm.at[cid, pl.ds(sid * RPS, RPS)])


R = 2000
GRID = N // R


def _dis_from(degT_ref):
    d = degT_ref[:, 0:1] + degT_ref[:, 1:2] + 1.0
    return lax.rsqrt(d)


def _lin_in_body(x_ref, degT_ref, w_ref, g_ref):
    dis = _dis_from(degT_ref)
    h = jnp.dot(x_ref[...], w_ref[...], preferred_element_type=jnp.float32)
    g_ref[...] = dis * h


_lin_in = pl.pallas_call(
    _lin_in_body,
    grid=(GRID,),
    in_specs=[
        pl.BlockSpec((R, D), lambda i: (i, 0)),
        pl.BlockSpec((R, 2), lambda i: (i, 0)),
        pl.BlockSpec((D, D), lambda i: (0, 0)),
    ],
    out_specs=pl.BlockSpec((R, D), lambda i: (i, 0)),
    out_shape=jax.ShapeDtypeStruct((N, D), jnp.float32),
)


def _mid_body(acc_ref, g1_ref, degT_ref, b1_ref, w_ref, g2_ref):
    dis = _dis_from(degT_ref)
    agg = acc_ref[0] + acc_ref[1] + g1_ref[...]
    h1 = jnp.maximum(dis * agg + b1_ref[...], 0.0)
    h2 = jnp.dot(h1, w_ref[...], preferred_element_type=jnp.float32)
    g2_ref[...] = dis * h2


_mid = pl.pallas_call(
    _mid_body,
    grid=(GRID,),
    in_specs=[
        pl.BlockSpec((NC, R, D), lambda i: (0, i, 0)),
        pl.BlockSpec((R, D), lambda i: (i, 0)),
        pl.BlockSpec((R, 2), lambda i: (i, 0)),
        pl.BlockSpec((1, D), lambda i: (0, 0)),
        pl.BlockSpec((D, D), lambda i: (0, 0)),
    ],
    out_specs=pl.BlockSpec((R, D), lambda i: (i, 0)),
    out_shape=jax.ShapeDtypeStruct((N, D), jnp.float32),
)


def _fin_body(acc_ref, g2_ref, degT_ref, b2_ref, out_ref):
    dis = _dis_from(degT_ref)
    agg = acc_ref[0] + acc_ref[1] + g2_ref[...]
    out_ref[...] = dis * agg + b2_ref[...]


_fin = pl.pallas_call(
    _fin_body,
    grid=(GRID,),
    in_specs=[
        pl.BlockSpec((NC, R, D), lambda i: (0, i, 0)),
        pl.BlockSpec((R, D), lambda i: (i, 0)),
        pl.BlockSpec((R, 2), lambda i: (i, 0)),
        pl.BlockSpec((1, D), lambda i: (0, 0)),
    ],
    out_specs=pl.BlockSpec((R, D), lambda i: (i, 0)),
    out_shape=jax.ShapeDtypeStruct((N, D), jnp.float32),
)


def kernel(x, edge_index, W1, b1, W2, b2):
    src3 = edge_index[0].reshape(NW, EPW)
    dst3 = edge_index[1].reshape(NW, CH, C)
    degp = _deg_sc(dst3)
    degT = degp.T
    g1 = _lin_in(x, degT, W1)
    acc1 = _scat_sc(g1, src3, dst3)
    g2 = _mid(acc1, g1, degT, b1.reshape(1, D), W2)
    acc2 = _scat_sc(g2, src3, dst3)
    out = _fin(acc2, g2, degT, b2.reshape(1, D))
    return out
